# Initial kernel scaffold; baseline (speedup 1.0000x reference)
import jax
import jax.numpy as jnp
from jax.experimental import pallas as pl


def kernel(h_V, h_E, src_idx, batch_id, dst_idx, W1_w, W1_b, W2_w, W2_b, W3_w, W3_b, A, ln1_g, ln1_b, ln2_g, ln2_b, Win_w, Win_b, Wout_w, Wout_b):
    raise NotImplementedError("write your pallas kernel here")


# speedup vs baseline: 3.2922x; 3.2922x over previous
import jax
import jax.numpy as jnp
from jax import lax
from jax.experimental import pallas as pl
from jax.experimental.pallas import tpu as pltpu
from jax.experimental.pallas import tpu_sc as plsc

N = 10000
E = 320000
H = 128
NC = 2
NS = 16
NW = NC * NS
EPW = E // NW
GC = 80
SC_CHUNK = 200
NPT = 624
NTAIL = N - NPT * NS
EB = 2000
NB = 1000


def _gelu(x):
    return 0.5 * x * (1.0 + lax.erf(x * 0.7071067811865476))


def _prep_body(hv_ref, wp_ref, wq_ref, a2_ref, p_ref, q_ref, as_ref, ad_ref):
    hv = hv_ref[...]
    p_ref[...] = jnp.dot(hv, wp_ref[...], preferred_element_type=jnp.float32)
    q_ref[...] = jnp.dot(hv, wq_ref[...], preferred_element_type=jnp.float32)
    al = lax.dot_general(a2_ref[...], hv, (((1,), (1,)), ((), ())),
                         preferred_element_type=jnp.float32)
    as_ref[...] = al[0:1, :]
    ad_ref[...] = al[1:2, :]


def _edge_body(he_ref, g_ref, lp_ref, w1e_ref, b1_ref, w2_ref, b2_ref, w3_ref,
               b3_ref, ae_ref, msg_ref, sum_ref):
    he = he_ref[...]
    x1 = g_ref[...] + jnp.dot(he, w1e_ref[...], preferred_element_type=jnp.float32) + b1_ref[...]
    x2 = jnp.dot(_gelu(x1), w2_ref[...], preferred_element_type=jnp.float32) + b2_ref[...]
    msg = jnp.dot(_gelu(x2), w3_ref[...], preferred_element_type=jnp.float32) + b3_ref[...]
    logit = lp_ref[...] + jnp.sum(he * ae_ref[...], axis=1, keepdims=True)
    leaky = jnp.where(logit >= 0, logit, 0.01 * logit)
    w = jnp.exp(1.0 / (1.0 + jnp.exp(-leaky)))

    @pl.when(pl.program_id(0) == 0)
    def _():
        sum_ref[0, 0] = 0.0

    sum_ref[0, 0] += jnp.sum(w)
    msg_ref[...] = msg * w


def _node_body(hv_ref, pa_ref, pb_ref, sum_ref, ln1g_ref, ln1b_ref, ln2g_ref,
               ln2b_ref, win_ref, winb_ref, wout_ref, woutb_ref, out_ref):
    scale = 1.0 / (30.0 * sum_ref[0, 0])
    x = hv_ref[...] + (pa_ref[...] + pb_ref[...]) * scale
    mu = jnp.mean(x, axis=1, keepdims=True)
    xc = x - mu
    var = jnp.mean(xc * xc, axis=1, keepdims=True)
    xn = xc * lax.rsqrt(var + 1e-5) * ln1g_ref[...] + ln1b_ref[...]
    y = jnp.dot(_gelu(jnp.dot(xn, win_ref[...], preferred_element_type=jnp.float32)
                      + winb_ref[...]),
                wout_ref[...], preferred_element_type=jnp.float32) + woutb_ref[...]
    z = xn + y
    mu2 = jnp.mean(z, axis=1, keepdims=True)
    zc = z - mu2
    var2 = jnp.mean(zc * zc, axis=1, keepdims=True)
    out_ref[...] = zc * lax.rsqrt(var2 + 1e-5) * ln2g_ref[...] + ln2b_ref[...]


def _sc_gather_body(p_hbm, q_hbm, as_hbm, ad_hbm, src_hbm, dst_hbm, g_hbm,
                    l_hbm, si_v, di_v, ps_v, qd_v, as_v, ad_v, gl_v, sem1, sem2):
    wid = lax.axis_index("s") * NC + lax.axis_index("c")
    pltpu.sync_copy(as_hbm.at[0], as_v)
    pltpu.sync_copy(ad_hbm.at[0], ad_v)

    def chunk(i, carry):
        base = wid * EPW + i * GC
        pltpu.sync_copy(src_hbm.at[pl.ds(base, GC)], si_v)
        pltpu.sync_copy(dst_hbm.at[pl.ds(base, GC)], di_v)
        cp1 = pltpu.async_copy(p_hbm.at[si_v], ps_v, sem1)
        cp2 = pltpu.async_copy(q_hbm.at[di_v], qd_v, sem2)
        cp1.wait()
        cp2.wait()

        def add_row(r, c2):
            for k in range(H // 16):
                sl = pl.ds(k * 16, 16)
                ps_v[r, sl] = ps_v[r, sl] + qd_v[r, sl]
            return c2

        lax.fori_loop(0, GC, add_row, 0)

        def lrow(r, c2):
            sl = pl.ds(r * 16, 16)
            va = plsc.load_gather(as_v, [si_v[sl]])
            vb = plsc.load_gather(ad_v, [di_v[sl]])
            gl_v[sl] = va + vb
            return c2

        lax.fori_loop(0, GC // 16, lrow, 0)
        pltpu.sync_copy(ps_v, g_hbm.at[pl.ds(base, GC)])
        pltpu.sync_copy(gl_v, l_hbm.at[pl.ds(base, GC)])
        return carry

    lax.fori_loop(0, EPW // GC, chunk, 0)


def _sc_scatter_body(msg_hbm, src_hbm, out_hbm, idx_v, msg_v, acc_sh, sem):
    cid = lax.axis_index("c")
    sid = lax.axis_index("s")
    wid = sid * NC + cid

    def zrow(r, c):
        for k in range(H // 16):
            msg_v[r, pl.ds(k * 16, 16)] = jnp.zeros((16,), jnp.float32)
        return c

    lax.fori_loop(0, SC_CHUNK, zrow, 0)
    for j in range(NPT // SC_CHUNK):
        pltpu.sync_copy(msg_v, acc_sh.at[pl.ds(sid * NPT + j * SC_CHUNK, SC_CHUNK)])
    pltpu.sync_copy(msg_v.at[pl.ds(0, NPT % SC_CHUNK)],
                    acc_sh.at[pl.ds(sid * NPT + (NPT // SC_CHUNK) * SC_CHUNK,
                                    NPT % SC_CHUNK)])

    @pl.when(sid == 0)
    def _():
        pltpu.sync_copy(msg_v.at[pl.ds(0, NTAIL)],
                        acc_sh.at[pl.ds(NPT * NS, NTAIL)])

    plsc.subcore_barrier()

    def chunk(i, carry):
        base = wid * EPW + i * SC_CHUNK
        pltpu.sync_copy(src_hbm.at[pl.ds(base, SC_CHUNK)], idx_v)
        pltpu.sync_copy(msg_hbm.at[pl.ds(base, SC_CHUNK)], msg_v)
        pltpu.sync_copy(msg_v, acc_sh.at[idx_v], add=True)
        return carry

    lax.fori_loop(0, EPW // SC_CHUNK, chunk, 0)
    plsc.subcore_barrier()
    pltpu.sync_copy(acc_sh.at[pl.ds(sid * NPT, NPT)],
                    out_hbm.at[cid].at[pl.ds(sid * NPT, NPT)])

    @pl.when(sid == 0)
    def _():
        pltpu.sync_copy(acc_sh.at[pl.ds(NPT * NS, NTAIL)],
                        out_hbm.at[cid].at[pl.ds(NPT * NS, NTAIL)])


def kernel(h_V, h_E, src_idx, batch_id, dst_idx, W1_w, W1_b, W2_w, W2_b, W3_w,
           W3_b, A, ln1_g, ln1_b, ln2_g, ln2_b, Win_w, Win_b, Wout_w, Wout_b):
    f32 = jnp.float32
    wp = W1_w[0:H]
    wq = W1_w[2 * H:3 * H]
    w1e = W1_w[H:2 * H]
    a2 = jnp.concatenate([A[0:H], A[2 * H:3 * H]], axis=1).T
    ae = A[H:2 * H].reshape(1, H)

    p_tab, q_tab, as_tab, ad_tab = pl.pallas_call(
        _prep_body,
        out_shape=[jax.ShapeDtypeStruct((N, H), f32),
                   jax.ShapeDtypeStruct((N, H), f32),
                   jax.ShapeDtypeStruct((1, N), f32),
                   jax.ShapeDtypeStruct((1, N), f32)],
    )(h_V, wp, wq, a2)

    sc_mesh = plsc.VectorSubcoreMesh(core_axis_name="c", subcore_axis_name="s")
    sc_params = pltpu.CompilerParams(needs_layout_passes=False)
    g_tab, l_tab = pl.kernel(
        _sc_gather_body,
        compiler_params=sc_params,
        out_type=[jax.ShapeDtypeStruct((E, H), f32),
                  jax.ShapeDtypeStruct((E,), f32)],
        mesh=sc_mesh,
        scratch_types=[
            pltpu.VMEM((GC,), jnp.int32),
            pltpu.VMEM((GC,), jnp.int32),
            pltpu.VMEM((GC, H), f32),
            pltpu.VMEM((GC, H), f32),
            pltpu.VMEM((N,), f32),
            pltpu.VMEM((N,), f32),
            pltpu.VMEM((GC,), f32),
            pltpu.SemaphoreType.DMA,
            pltpu.SemaphoreType.DMA,
        ],
    )(p_tab, q_tab, as_tab, ad_tab, src_idx, dst_idx)

    nb_e = E // EB
    msg, wsum = pl.pallas_call(
        _edge_body,
        grid=(nb_e,),
        in_specs=[
            pl.BlockSpec((EB, H), lambda i: (i, 0)),
            pl.BlockSpec((EB, H), lambda i: (i, 0)),
            pl.BlockSpec((EB, 1), lambda i: (i, 0)),
            pl.BlockSpec((H, H), lambda i: (0, 0)),
            pl.BlockSpec((1, H), lambda i: (0, 0)),
            pl.BlockSpec((H, H), lambda i: (0, 0)),
            pl.BlockSpec((1, H), lambda i: (0, 0)),
            pl.BlockSpec((H, H), lambda i: (0, 0)),
            pl.BlockSpec((1, H), lambda i: (0, 0)),
            pl.BlockSpec((1, H), lambda i: (0, 0)),
        ],
        out_specs=[
            pl.BlockSpec((EB, H), lambda i: (i, 0)),
            pl.BlockSpec(memory_space=pltpu.SMEM),
        ],
        out_shape=[jax.ShapeDtypeStruct((E, H), f32),
                   jax.ShapeDtypeStruct((1, 1), f32)],
    )(h_E, g_tab, l_tab.reshape(E, 1), w1e, W1_b.reshape(1, H), W2_w,
      W2_b.reshape(1, H), W3_w, W3_b.reshape(1, H), ae)

    parts = pl.kernel(
        _sc_scatter_body,
        compiler_params=sc_params,
        out_type=jax.ShapeDtypeStruct((NC, N, H), f32),
        mesh=sc_mesh,
        scratch_types=[
            pltpu.VMEM((SC_CHUNK,), jnp.int32),
            pltpu.VMEM((SC_CHUNK, H), f32),
            pltpu.VMEM_SHARED((N, H), f32),
            pltpu.SemaphoreType.DMA,
        ],
    )(msg, src_idx)

    nb_n = N // NB
    out = pl.pallas_call(
        _node_body,
        grid=(nb_n,),
        in_specs=[
            pl.BlockSpec((NB, H), lambda i: (i, 0)),
            pl.BlockSpec((NB, H), lambda i: (i, 0)),
            pl.BlockSpec((NB, H), lambda i: (i, 0)),
            pl.BlockSpec(memory_space=pltpu.SMEM),
            pl.BlockSpec((1, H), lambda i: (0, 0)),
            pl.BlockSpec((1, H), lambda i: (0, 0)),
            pl.BlockSpec((1, H), lambda i: (0, 0)),
            pl.BlockSpec((1, H), lambda i: (0, 0)),
            pl.BlockSpec((H, 4 * H), lambda i: (0, 0)),
            pl.BlockSpec((1, 4 * H), lambda i: (0, 0)),
            pl.BlockSpec((4 * H, H), lambda i: (0, 0)),
            pl.BlockSpec((1, H), lambda i: (0, 0)),
        ],
        out_specs=pl.BlockSpec((NB, H), lambda i: (i, 0)),
        out_shape=jax.ShapeDtypeStruct((N, H), f32),
    )(h_V, parts[0], parts[1], wsum, ln1_g.reshape(1, H), ln1_b.reshape(1, H),
      ln2_g.reshape(1, H), ln2_b.reshape(1, H), Win_w, Win_b.reshape(1, 4 * H),
      Wout_w, Wout_b.reshape(1, H))
    return out
